# baseline (speedup 1.0000x reference)
import functools

import jax
import jax.numpy as jnp
from jax import lax
from jax.experimental import pallas as pl
from jax.experimental.pallas import tpu as pltpu
from jax.experimental.pallas import tpu_sc as plsc

BATCH = 16384
EMBED = 64
NWORKERS = 32
BPW = BATCH // NWORKERS
IDX_CHUNK = 128
NCHUNK = BPW // IDX_CHUNK


def _sc_gather(upair_idx, ipair_idx, ut2, it2):
    mesh = plsc.VectorSubcoreMesh(core_axis_name="c", subcore_axis_name="s")

    @functools.partial(
        pl.kernel,
        mesh=mesh,
        out_type=(
            jax.ShapeDtypeStruct((BATCH, 128), jnp.float32),
            jax.ShapeDtypeStruct((BATCH, 128), jnp.float32),
        ),
        scratch_types=[
            pltpu.VMEM((NCHUNK, IDX_CHUNK), jnp.int32),
            pltpu.VMEM((NCHUNK, IDX_CHUNK), jnp.int32),
            pltpu.VMEM((BPW, 128), jnp.float32),
            pltpu.SemaphoreType.DMA,
            pltpu.SemaphoreType.DMA,
        ],
    )
    def gather_kernel(uidx_hbm, iidx_hbm, ut_hbm, it_hbm, uout_hbm, iout_hbm,
                      uidx_v, iidx_v, rows_v, gsem, osem):
        wid = lax.axis_index("s") * 2 + lax.axis_index("c")
        base = wid * BPW
        idx_row0 = wid * NCHUNK
        pltpu.sync_copy(uidx_hbm.at[pl.ds(idx_row0, NCHUNK), :], uidx_v)
        pltpu.sync_copy(iidx_hbm.at[pl.ds(idx_row0, NCHUNK), :], iidx_v)
        for tbl_hbm, idx_v, out_hbm in (
                (ut_hbm, uidx_v, uout_hbm), (it_hbm, iidx_v, iout_hbm)):
            copies = []
            for j in range(NCHUNK):
                dst = rows_v.at[pl.ds(j * IDX_CHUNK, IDX_CHUNK), :]
                copies.append(pltpu.async_copy(tbl_hbm.at[idx_v.at[j]], dst, gsem))
            for c in copies:
                c.wait()
            pltpu.sync_copy(rows_v, out_hbm.at[pl.ds(base, BPW), :])

    return gather_kernel(upair_idx, ipair_idx, ut2, it2)


def _mlp_body(up_ref, ip_ref, pu_ref, pi_ref, w1a_ref, w1b_ref, b1_ref,
              w2_ref, b2_ref, w3_ref, b3_ref, out_ref):
    up = up_ref[...]
    ip = ip_ref[...]
    ue = jnp.where(pu_ref[...] > 0.5, up[:, EMBED:], up[:, :EMBED])
    ie = jnp.where(pi_ref[...] > 0.5, ip[:, EMBED:], ip[:, :EMBED])
    h1 = jnp.dot(ue, w1a_ref[...], preferred_element_type=jnp.float32)
    h1 += jnp.dot(ie, w1b_ref[...], preferred_element_type=jnp.float32)
    h1 = jnp.maximum(h1 + b1_ref[...], 0.0)
    h2 = jnp.dot(h1, w2_ref[...], preferred_element_type=jnp.float32)
    h2 = jnp.maximum(h2 + b2_ref[...], 0.0)
    logit = jnp.dot(h2, w3_ref[...], preferred_element_type=jnp.float32)
    logit = logit + b3_ref[...]
    out_ref[...] = 1.0 / (1.0 + jnp.exp(-logit))


def _mlp(upair, ipair, upar, ipar, w1a, w1b, b1, w2, b2, w3, b3,
         interpret=False):
    BM = 2048
    grid = (BATCH // BM,)

    def full(shape):
        return pl.BlockSpec(shape, lambda i: (0, 0))

    return pl.pallas_call(
        _mlp_body,
        grid=grid,
        in_specs=[
            pl.BlockSpec((BM, 128), lambda i: (i, 0)),
            pl.BlockSpec((BM, 128), lambda i: (i, 0)),
            pl.BlockSpec((BM, 1), lambda i: (i, 0)),
            pl.BlockSpec((BM, 1), lambda i: (i, 0)),
            full((EMBED, 128)),
            full((EMBED, 128)),
            full((1, 128)),
            full((128, EMBED)),
            full((1, EMBED)),
            full((EMBED, 1)),
            full((1, 1)),
        ],
        out_specs=pl.BlockSpec((BM, 1), lambda i: (i, 0)),
        out_shape=jax.ShapeDtypeStruct((BATCH, 1), jnp.float32),
        interpret=interpret,
    )(upair, ipair, upar, ipar, w1a, w1b, b1, w2, b2, w3, b3)


def kernel(user_ids, item_ids, user_table, item_table, W1, b1, W2, b2, W3, b3):
    uids = user_ids.astype(jnp.int32)
    iids = item_ids.astype(jnp.int32)
    upair_idx = (uids >> 1).reshape(BATCH // IDX_CHUNK, IDX_CHUNK)
    ipair_idx = (iids >> 1).reshape(BATCH // IDX_CHUNK, IDX_CHUNK)
    upar = (uids & 1).astype(jnp.float32).reshape(BATCH, 1)
    ipar = (iids & 1).astype(jnp.float32).reshape(BATCH, 1)
    ut2 = user_table.reshape(-1, 128)
    it2 = item_table.reshape(-1, 128)
    upair, ipair = _sc_gather(upair_idx, ipair_idx, ut2, it2)
    w1a = W1[:, :EMBED].T
    w1b = W1[:, EMBED:].T
    return _mlp(upair, ipair, upar, ipar, w1a, w1b, b1.reshape(1, 128),
                W2.T, b2.reshape(1, EMBED), W3.T, b3.reshape(1, 1))


# speedup vs baseline: 2.3561x; 2.3561x over previous
import functools
import math

import jax
import jax.numpy as jnp
from jax import lax
from jax.experimental import pallas as pl
from jax.experimental.pallas import tpu as pltpu
from jax.experimental.pallas import tpu_sc as plsc

BATCH = 16384
EMBED = 64
NROWS = 1000000
NWORKERS = 32
BPW = BATCH // NWORKERS
IDX_CHUNK = 128
NCHUNK = BPW // IDX_CHUNK
CVT_BM = 8192


def _cvt_body(u_ref, i_ref, o_ref):
    o_ref[...] = jnp.concatenate([u_ref[...].T, i_ref[...].T], axis=1)


def _convert(utT, itT):
    grid = (math.ceil(NROWS / CVT_BM),)
    return pl.pallas_call(
        _cvt_body,
        grid=grid,
        in_specs=[
            pl.BlockSpec((EMBED, CVT_BM), lambda i: (0, i)),
            pl.BlockSpec((EMBED, CVT_BM), lambda i: (0, i)),
        ],
        out_specs=pl.BlockSpec((CVT_BM, 128), lambda i: (i, 0)),
        out_shape=jax.ShapeDtypeStruct((NROWS, 128), jnp.float32),
    )(utT, itT)


def _sc_gather(uids2d, iids2d, table):
    mesh = plsc.VectorSubcoreMesh(core_axis_name="c", subcore_axis_name="s")

    @functools.partial(
        pl.kernel,
        mesh=mesh,
        out_type=(
            jax.ShapeDtypeStruct((BATCH, 128), jnp.float32),
            jax.ShapeDtypeStruct((BATCH, 128), jnp.float32),
        ),
        scratch_types=[
            pltpu.VMEM((NCHUNK, IDX_CHUNK), jnp.int32),
            pltpu.VMEM((NCHUNK, IDX_CHUNK), jnp.int32),
            pltpu.VMEM((BPW, 128), jnp.float32),
            pltpu.SemaphoreType.DMA,
        ],
    )
    def gather_kernel(uidx_hbm, iidx_hbm, tbl_hbm, uout_hbm, iout_hbm,
                      uidx_v, iidx_v, rows_v, sem):
        wid = lax.axis_index("s") * 2 + lax.axis_index("c")
        base = wid * BPW
        idx_row0 = wid * NCHUNK
        pltpu.sync_copy(uidx_hbm.at[pl.ds(idx_row0, NCHUNK), :], uidx_v)
        pltpu.sync_copy(iidx_hbm.at[pl.ds(idx_row0, NCHUNK), :], iidx_v)
        for idx_v, out_hbm in ((uidx_v, uout_hbm), (iidx_v, iout_hbm)):
            copies = []
            for j in range(NCHUNK):
                dst = rows_v.at[pl.ds(j * IDX_CHUNK, IDX_CHUNK), :]
                copies.append(pltpu.async_copy(tbl_hbm.at[idx_v.at[j]], dst, sem))
            for c in copies:
                c.wait()
            pltpu.sync_copy(rows_v, out_hbm.at[pl.ds(base, BPW), :])

    return gather_kernel(uids2d, iids2d, table)


def _mlp_body(u_ref, i_ref, w1a_ref, w1b_ref, b1_ref, w2_ref, b2_ref,
              w3_ref, b3_ref, out_ref):
    ue = u_ref[...][:, :EMBED]
    ie = i_ref[...][:, EMBED:]
    h1 = jnp.dot(ue, w1a_ref[...], preferred_element_type=jnp.float32)
    h1 += jnp.dot(ie, w1b_ref[...], preferred_element_type=jnp.float32)
    h1 = jnp.maximum(h1 + b1_ref[...], 0.0)
    h2 = jnp.dot(h1, w2_ref[...], preferred_element_type=jnp.float32)
    h2 = jnp.maximum(h2 + b2_ref[...], 0.0)
    logit = jnp.dot(h2, w3_ref[...], preferred_element_type=jnp.float32)
    logit = logit + b3_ref[...]
    out_ref[...] = 1.0 / (1.0 + jnp.exp(-logit))


def _mlp(urows, irows, w1a, w1b, b1, w2, b2, w3, b3, interpret=False):
    BM = 2048
    grid = (BATCH // BM,)

    def full(shape):
        return pl.BlockSpec(shape, lambda i: (0, 0))

    return pl.pallas_call(
        _mlp_body,
        grid=grid,
        in_specs=[
            pl.BlockSpec((BM, 128), lambda i: (i, 0)),
            pl.BlockSpec((BM, 128), lambda i: (i, 0)),
            full((EMBED, 128)),
            full((EMBED, 128)),
            full((1, 128)),
            full((128, EMBED)),
            full((1, EMBED)),
            full((EMBED, 1)),
            full((1, 1)),
        ],
        out_specs=pl.BlockSpec((BM, 1), lambda i: (i, 0)),
        out_shape=jax.ShapeDtypeStruct((BATCH, 1), jnp.float32),
        interpret=interpret,
    )(urows, irows, w1a, w1b, b1, w2, b2, w3, b3)


def kernel(user_ids, item_ids, user_table, item_table, W1, b1, W2, b2, W3, b3):
    uids2d = user_ids.astype(jnp.int32).reshape(BATCH // IDX_CHUNK, IDX_CHUNK)
    iids2d = item_ids.astype(jnp.int32).reshape(BATCH // IDX_CHUNK, IDX_CHUNK)
    table = _convert(user_table.T, item_table.T)
    urows, irows = _sc_gather(uids2d, iids2d, table)
    w1a = W1[:, :EMBED].T
    w1b = W1[:, EMBED:].T
    return _mlp(urows, irows, w1a, w1b, b1.reshape(1, 128),
                W2.T, b2.reshape(1, EMBED), W3.T, b3.reshape(1, 1))


# speedup vs baseline: 2.6607x; 1.1293x over previous
import functools
import math

import jax
import jax.numpy as jnp
from jax import lax
from jax.experimental import pallas as pl
from jax.experimental.pallas import tpu as pltpu
from jax.experimental.pallas import tpu_sc as plsc

BATCH = 16384
EMBED = 64
NROWS = 1000000
NWORKERS = 32
BPW = BATCH // NWORKERS
IDX_CHUNK = 128
NCHUNK = BPW // IDX_CHUNK
CVT_BM = 8192


def _cvt_body(u_ref, i_ref, eu_ref, ei_ref, o_ref):
    dn = (((0,), (0,)), ((), ()))
    o_ref[...] = (
        lax.dot_general(u_ref[...], eu_ref[...], dn,
                        preferred_element_type=jnp.float32)
        + lax.dot_general(i_ref[...], ei_ref[...], dn,
                          preferred_element_type=jnp.float32)
    )


def _convert(utT, itT):
    eu = jnp.eye(EMBED, 128, dtype=jnp.float32)
    ei = jnp.eye(EMBED, 128, k=EMBED, dtype=jnp.float32)
    grid = (math.ceil(NROWS / CVT_BM),)

    def full(shape):
        return pl.BlockSpec(shape, lambda i: (0, 0))

    return pl.pallas_call(
        _cvt_body,
        grid=grid,
        in_specs=[
            pl.BlockSpec((EMBED, CVT_BM), lambda i: (0, i)),
            pl.BlockSpec((EMBED, CVT_BM), lambda i: (0, i)),
            full((EMBED, 128)),
            full((EMBED, 128)),
        ],
        out_specs=pl.BlockSpec((CVT_BM, 128), lambda i: (i, 0)),
        out_shape=jax.ShapeDtypeStruct((NROWS, 128), jnp.float32),
        compiler_params=pltpu.CompilerParams(fuse_transposed_lhs_in_matmul=True),
    )(utT, itT, eu, ei)


def _sc_gather(uids2d, iids2d, table):
    mesh = plsc.VectorSubcoreMesh(core_axis_name="c", subcore_axis_name="s")

    @functools.partial(
        pl.kernel,
        mesh=mesh,
        out_type=(
            jax.ShapeDtypeStruct((BATCH, 128), jnp.float32),
            jax.ShapeDtypeStruct((BATCH, 128), jnp.float32),
        ),
        scratch_types=[
            pltpu.VMEM((NCHUNK, IDX_CHUNK), jnp.int32),
            pltpu.VMEM((NCHUNK, IDX_CHUNK), jnp.int32),
            pltpu.VMEM((BPW, 128), jnp.float32),
            pltpu.SemaphoreType.DMA,
        ],
    )
    def gather_kernel(uidx_hbm, iidx_hbm, tbl_hbm, uout_hbm, iout_hbm,
                      uidx_v, iidx_v, rows_v, sem):
        wid = lax.axis_index("s") * 2 + lax.axis_index("c")
        base = wid * BPW
        idx_row0 = wid * NCHUNK
        pltpu.sync_copy(uidx_hbm.at[pl.ds(idx_row0, NCHUNK), :], uidx_v)
        pltpu.sync_copy(iidx_hbm.at[pl.ds(idx_row0, NCHUNK), :], iidx_v)
        for idx_v, out_hbm in ((uidx_v, uout_hbm), (iidx_v, iout_hbm)):
            copies = []
            for j in range(NCHUNK):
                dst = rows_v.at[pl.ds(j * IDX_CHUNK, IDX_CHUNK), :]
                copies.append(pltpu.async_copy(tbl_hbm.at[idx_v.at[j]], dst, sem))
            for c in copies:
                c.wait()
            pltpu.sync_copy(rows_v, out_hbm.at[pl.ds(base, BPW), :])

    return gather_kernel(uids2d, iids2d, table)


def _mlp_body(u_ref, i_ref, w1a_ref, w1b_ref, b1_ref, w2_ref, b2_ref,
              w3_ref, b3_ref, out_ref):
    ue = u_ref[...][:, :EMBED]
    ie = i_ref[...][:, EMBED:]
    h1 = jnp.dot(ue, w1a_ref[...], preferred_element_type=jnp.float32)
    h1 += jnp.dot(ie, w1b_ref[...], preferred_element_type=jnp.float32)
    h1 = jnp.maximum(h1 + b1_ref[...], 0.0)
    h2 = jnp.dot(h1, w2_ref[...], preferred_element_type=jnp.float32)
    h2 = jnp.maximum(h2 + b2_ref[...], 0.0)
    logit = jnp.dot(h2, w3_ref[...], preferred_element_type=jnp.float32)
    logit = logit + b3_ref[...]
    out_ref[...] = 1.0 / (1.0 + jnp.exp(-logit))


def _mlp(urows, irows, w1a, w1b, b1, w2, b2, w3, b3, interpret=False):
    BM = 2048
    grid = (BATCH // BM,)

    def full(shape):
        return pl.BlockSpec(shape, lambda i: (0, 0))

    return pl.pallas_call(
        _mlp_body,
        grid=grid,
        in_specs=[
            pl.BlockSpec((BM, 128), lambda i: (i, 0)),
            pl.BlockSpec((BM, 128), lambda i: (i, 0)),
            full((EMBED, 128)),
            full((EMBED, 128)),
            full((1, 128)),
            full((128, EMBED)),
            full((1, EMBED)),
            full((EMBED, 1)),
            full((1, 1)),
        ],
        out_specs=pl.BlockSpec((BM, 1), lambda i: (i, 0)),
        out_shape=jax.ShapeDtypeStruct((BATCH, 1), jnp.float32),
        interpret=interpret,
    )(urows, irows, w1a, w1b, b1, w2, b2, w3, b3)


def kernel(user_ids, item_ids, user_table, item_table, W1, b1, W2, b2, W3, b3):
    uids2d = user_ids.astype(jnp.int32).reshape(BATCH // IDX_CHUNK, IDX_CHUNK)
    iids2d = item_ids.astype(jnp.int32).reshape(BATCH // IDX_CHUNK, IDX_CHUNK)
    table = _convert(user_table.T, item_table.T)
    urows, irows = _sc_gather(uids2d, iids2d, table)
    w1a = W1[:, :EMBED].T
    w1b = W1[:, EMBED:].T
    return _mlp(urows, irows, w1a, w1b, b1.reshape(1, 128),
                W2.T, b2.reshape(1, EMBED), W3.T, b3.reshape(1, 1))


# speedup vs baseline: 2.8704x; 1.0788x over previous
import functools
import math

import jax
import jax.numpy as jnp
from jax import lax
from jax.experimental import pallas as pl
from jax.experimental.pallas import tpu as pltpu
from jax.experimental.pallas import tpu_sc as plsc

BATCH = 16384
EMBED = 64
NROWS = 1000000
NWORKERS = 32
BPW = BATCH // NWORKERS
IDX_CHUNK = 128
NCHUNK = BPW // IDX_CHUNK
CVT_BM = 8192


def _cvt_body(u_ref, i_ref, eu_ref, ei_ref, o_ref):
    dn = (((0,), (0,)), ((), ()))
    c = (
        lax.dot_general(u_ref[...], eu_ref[...], dn,
                        preferred_element_type=jnp.float32)
        + lax.dot_general(i_ref[...], ei_ref[...], dn,
                          preferred_element_type=jnp.float32)
    )
    o_ref[...] = pltpu.bitcast(c.astype(jnp.bfloat16), jnp.int32)


def _convert(utT, itT):
    eu = jnp.eye(EMBED, 128, dtype=jnp.float32)
    ei = jnp.eye(EMBED, 128, k=EMBED, dtype=jnp.float32)
    grid = (math.ceil(NROWS / CVT_BM),)

    def full(shape):
        return pl.BlockSpec(shape, lambda i: (0, 0))

    return pl.pallas_call(
        _cvt_body,
        grid=grid,
        in_specs=[
            pl.BlockSpec((EMBED, CVT_BM), lambda i: (0, i)),
            pl.BlockSpec((EMBED, CVT_BM), lambda i: (0, i)),
            full((EMBED, 128)),
            full((EMBED, 128)),
        ],
        out_specs=pl.BlockSpec((CVT_BM // 2, 128), lambda i: (i, 0)),
        out_shape=jax.ShapeDtypeStruct((NROWS // 2, 128), jnp.int32),
        compiler_params=pltpu.CompilerParams(fuse_transposed_lhs_in_matmul=True),
    )(utT, itT, eu, ei)


def _sc_gather(uids2d, iids2d, table):
    mesh = plsc.VectorSubcoreMesh(core_axis_name="c", subcore_axis_name="s")

    @functools.partial(
        pl.kernel,
        mesh=mesh,
        out_type=(
            jax.ShapeDtypeStruct((BATCH, 128), jnp.int32),
            jax.ShapeDtypeStruct((BATCH, 128), jnp.int32),
        ),
        scratch_types=[
            pltpu.VMEM((NCHUNK, IDX_CHUNK), jnp.int32),
            pltpu.VMEM((NCHUNK, IDX_CHUNK), jnp.int32),
            pltpu.VMEM((BPW, 128), jnp.int32),
            pltpu.SemaphoreType.DMA,
        ],
    )
    def gather_kernel(uidx_hbm, iidx_hbm, tbl_hbm, uout_hbm, iout_hbm,
                      uidx_v, iidx_v, rows_v, sem):
        wid = lax.axis_index("s") * 2 + lax.axis_index("c")
        base = wid * BPW
        idx_row0 = wid * NCHUNK
        pltpu.sync_copy(uidx_hbm.at[pl.ds(idx_row0, NCHUNK), :], uidx_v)
        pltpu.sync_copy(iidx_hbm.at[pl.ds(idx_row0, NCHUNK), :], iidx_v)
        for idx_v, out_hbm in ((uidx_v, uout_hbm), (iidx_v, iout_hbm)):
            copies = []
            for j in range(NCHUNK):
                dst = rows_v.at[pl.ds(j * IDX_CHUNK, IDX_CHUNK), :]
                copies.append(pltpu.async_copy(tbl_hbm.at[idx_v.at[j]], dst, sem))
            for c in copies:
                c.wait()
            pltpu.sync_copy(rows_v, out_hbm.at[pl.ds(base, BPW), :])

    return gather_kernel(uids2d, iids2d, table)


def _unpack(w, par):
    odd = par > 0
    bits = jnp.where(odd, w & jnp.int32(-65536), w << 16)
    return pltpu.bitcast(bits, jnp.float32)


def _mlp_body(u_ref, i_ref, pu_ref, pi_ref, w1a_ref, w1b_ref, b1_ref,
              w2_ref, b2_ref, w3_ref, b3_ref, out_ref):
    ue = _unpack(u_ref[...], pu_ref[...])[:, :EMBED]
    ie = _unpack(i_ref[...], pi_ref[...])[:, EMBED:]
    h1 = jnp.dot(ue, w1a_ref[...], preferred_element_type=jnp.float32)
    h1 += jnp.dot(ie, w1b_ref[...], preferred_element_type=jnp.float32)
    h1 = jnp.maximum(h1 + b1_ref[...], 0.0)
    h2 = jnp.dot(h1, w2_ref[...], preferred_element_type=jnp.float32)
    h2 = jnp.maximum(h2 + b2_ref[...], 0.0)
    logit = jnp.dot(h2, w3_ref[...], preferred_element_type=jnp.float32)
    logit = logit + b3_ref[...]
    out_ref[...] = 1.0 / (1.0 + jnp.exp(-logit))


def _mlp(urows, irows, upar, ipar, w1a, w1b, b1, w2, b2, w3, b3,
         interpret=False):
    BM = 2048
    grid = (BATCH // BM,)

    def full(shape):
        return pl.BlockSpec(shape, lambda i: (0, 0))

    return pl.pallas_call(
        _mlp_body,
        grid=grid,
        in_specs=[
            pl.BlockSpec((BM, 128), lambda i: (i, 0)),
            pl.BlockSpec((BM, 128), lambda i: (i, 0)),
            pl.BlockSpec((BM, 1), lambda i: (i, 0)),
            pl.BlockSpec((BM, 1), lambda i: (i, 0)),
            full((EMBED, 128)),
            full((EMBED, 128)),
            full((1, 128)),
            full((128, EMBED)),
            full((1, EMBED)),
            full((EMBED, 1)),
            full((1, 1)),
        ],
        out_specs=pl.BlockSpec((BM, 1), lambda i: (i, 0)),
        out_shape=jax.ShapeDtypeStruct((BATCH, 1), jnp.float32),
        interpret=interpret,
    )(urows, irows, upar, ipar, w1a, w1b, b1, w2, b2, w3, b3)


def kernel(user_ids, item_ids, user_table, item_table, W1, b1, W2, b2, W3, b3):
    uids = user_ids.astype(jnp.int32)
    iids = item_ids.astype(jnp.int32)
    uidx2d = (uids >> 1).reshape(BATCH // IDX_CHUNK, IDX_CHUNK)
    iidx2d = (iids >> 1).reshape(BATCH // IDX_CHUNK, IDX_CHUNK)
    upar = (uids & 1).reshape(BATCH, 1)
    ipar = (iids & 1).reshape(BATCH, 1)
    table = _convert(user_table.T, item_table.T)
    urows, irows = _sc_gather(uidx2d, iidx2d, table)
    w1a = W1[:, :EMBED].T
    w1b = W1[:, EMBED:].T
    return _mlp(urows, irows, upar, ipar, w1a, w1b, b1.reshape(1, 128),
                W2.T, b2.reshape(1, EMBED), W3.T, b3.reshape(1, 1))


# speedup vs baseline: 3.1781x; 1.1072x over previous
import functools
import math

import jax
import jax.numpy as jnp
from jax import lax
from jax.experimental import pallas as pl
from jax.experimental.pallas import tpu as pltpu
from jax.experimental.pallas import tpu_sc as plsc

BATCH = 16384
EMBED = 64
NROWS = 1000000
NWORKERS = 32
BPW = BATCH // NWORKERS
IDX_CHUNK = 128
NCHUNK = BPW // IDX_CHUNK
CVT_BM = 16384


def _cvt_body(u_ref, i_ref, eu_ref, ei_ref, o_ref):
    dn = (((0,), (0,)), ((), ()))
    c = (
        lax.dot_general(u_ref[...], eu_ref[...], dn,
                        preferred_element_type=jnp.float32)
        + lax.dot_general(i_ref[...], ei_ref[...], dn,
                          preferred_element_type=jnp.float32)
    )
    o_ref[...] = pltpu.bitcast(c.astype(jnp.bfloat16), jnp.int32)


def _convert(utT, itT):
    eu = jnp.eye(EMBED, 128, dtype=jnp.float32)
    ei = jnp.eye(EMBED, 128, k=EMBED, dtype=jnp.float32)
    grid = (math.ceil(NROWS / CVT_BM),)

    def full(shape):
        return pl.BlockSpec(shape, lambda i: (0, 0))

    return pl.pallas_call(
        _cvt_body,
        grid=grid,
        in_specs=[
            pl.BlockSpec((EMBED, CVT_BM), lambda i: (0, i)),
            pl.BlockSpec((EMBED, CVT_BM), lambda i: (0, i)),
            full((EMBED, 128)),
            full((EMBED, 128)),
        ],
        out_specs=pl.BlockSpec((CVT_BM // 2, 128), lambda i: (i, 0)),
        out_shape=jax.ShapeDtypeStruct((NROWS // 2, 128), jnp.int32),
        compiler_params=pltpu.CompilerParams(fuse_transposed_lhs_in_matmul=True),
    )(utT, itT, eu, ei)


def _sc_gather(uids2d, iids2d, table):
    mesh = plsc.VectorSubcoreMesh(core_axis_name="c", subcore_axis_name="s")

    @functools.partial(
        pl.kernel,
        mesh=mesh,
        out_type=(
            jax.ShapeDtypeStruct((BATCH, 128), jnp.int32),
            jax.ShapeDtypeStruct((BATCH, 128), jnp.int32),
        ),
        scratch_types=[
            pltpu.VMEM((NCHUNK, IDX_CHUNK), jnp.int32),
            pltpu.VMEM((NCHUNK, IDX_CHUNK), jnp.int32),
            pltpu.VMEM((BPW, 128), jnp.int32),
            pltpu.SemaphoreType.DMA,
        ],
    )
    def gather_kernel(uidx_hbm, iidx_hbm, tbl_hbm, uout_hbm, iout_hbm,
                      uidx_v, iidx_v, rows_v, sem):
        wid = lax.axis_index("s") * 2 + lax.axis_index("c")
        base = wid * BPW
        idx_row0 = wid * NCHUNK
        pltpu.sync_copy(uidx_hbm.at[pl.ds(idx_row0, NCHUNK), :], uidx_v)
        pltpu.sync_copy(iidx_hbm.at[pl.ds(idx_row0, NCHUNK), :], iidx_v)
        for idx_v, out_hbm in ((uidx_v, uout_hbm), (iidx_v, iout_hbm)):
            copies = []
            for j in range(NCHUNK):
                dst = rows_v.at[pl.ds(j * IDX_CHUNK, IDX_CHUNK), :]
                copies.append(pltpu.async_copy(tbl_hbm.at[idx_v.at[j]], dst, sem))
            for c in copies:
                c.wait()
            pltpu.sync_copy(rows_v, out_hbm.at[pl.ds(base, BPW), :])

    return gather_kernel(uids2d, iids2d, table)


def _unpack(w, par):
    odd = par > 0
    bits = jnp.where(odd, w & jnp.int32(-65536), w << 16)
    return pltpu.bitcast(bits, jnp.float32)


def _mlp_body(u_ref, i_ref, pu_ref, pi_ref, w1a_ref, w1b_ref, b1_ref,
              w2_ref, b2_ref, w3_ref, b3_ref, out_ref):
    ue = _unpack(u_ref[...], pu_ref[...])[:, :EMBED]
    ie = _unpack(i_ref[...], pi_ref[...])[:, EMBED:]
    h1 = jnp.dot(ue, w1a_ref[...], preferred_element_type=jnp.float32)
    h1 += jnp.dot(ie, w1b_ref[...], preferred_element_type=jnp.float32)
    h1 = jnp.maximum(h1 + b1_ref[...], 0.0)
    h2 = jnp.dot(h1, w2_ref[...], preferred_element_type=jnp.float32)
    h2 = jnp.maximum(h2 + b2_ref[...], 0.0)
    logit = jnp.dot(h2, w3_ref[...], preferred_element_type=jnp.float32)
    logit = logit + b3_ref[...]
    out_ref[...] = 1.0 / (1.0 + jnp.exp(-logit))


def _mlp(urows, irows, upar, ipar, w1a, w1b, b1, w2, b2, w3, b3,
         interpret=False):
    BM = 2048
    grid = (BATCH // BM,)

    def full(shape):
        return pl.BlockSpec(shape, lambda i: (0, 0))

    return pl.pallas_call(
        _mlp_body,
        grid=grid,
        in_specs=[
            pl.BlockSpec((BM, 128), lambda i: (i, 0)),
            pl.BlockSpec((BM, 128), lambda i: (i, 0)),
            pl.BlockSpec((BM, 1), lambda i: (i, 0)),
            pl.BlockSpec((BM, 1), lambda i: (i, 0)),
            full((EMBED, 128)),
            full((EMBED, 128)),
            full((1, 128)),
            full((128, EMBED)),
            full((1, EMBED)),
            full((EMBED, 1)),
            full((1, 1)),
        ],
        out_specs=pl.BlockSpec((BM, 1), lambda i: (i, 0)),
        out_shape=jax.ShapeDtypeStruct((BATCH, 1), jnp.float32),
        interpret=interpret,
    )(urows, irows, upar, ipar, w1a, w1b, b1, w2, b2, w3, b3)


def kernel(user_ids, item_ids, user_table, item_table, W1, b1, W2, b2, W3, b3):
    uids = user_ids.astype(jnp.int32)
    iids = item_ids.astype(jnp.int32)
    uidx2d = (uids >> 1).reshape(BATCH // IDX_CHUNK, IDX_CHUNK)
    iidx2d = (iids >> 1).reshape(BATCH // IDX_CHUNK, IDX_CHUNK)
    upar = (uids & 1).reshape(BATCH, 1)
    ipar = (iids & 1).reshape(BATCH, 1)
    table = _convert(user_table.T, item_table.T)
    urows, irows = _sc_gather(uidx2d, iidx2d, table)
    w1a = W1[:, :EMBED].T
    w1b = W1[:, EMBED:].T
    return _mlp(urows, irows, upar, ipar, w1a, w1b, b1.reshape(1, 128),
                W2.T, b2.reshape(1, EMBED), W3.T, b3.reshape(1, 1))


# speedup vs baseline: 3.3246x; 1.0461x over previous
import functools
import math

import jax
import jax.numpy as jnp
from jax import lax
from jax.experimental import pallas as pl
from jax.experimental.pallas import tpu as pltpu
from jax.experimental.pallas import tpu_sc as plsc

BATCH = 16384
EMBED = 64
NROWS = 1000000
NWORKERS = 32
BPW = BATCH // NWORKERS
IDX_CHUNK = 128
NCHUNK = BPW // IDX_CHUNK
CVT_BM = 32768


def _cvt_body(u_ref, i_ref, eu_ref, ei_ref, o_ref):
    dn = (((0,), (0,)), ((), ()))
    c = (
        lax.dot_general(u_ref[...], eu_ref[...], dn,
                        preferred_element_type=jnp.float32)
        + lax.dot_general(i_ref[...], ei_ref[...], dn,
                          preferred_element_type=jnp.float32)
    )
    o_ref[...] = pltpu.bitcast(c.astype(jnp.bfloat16), jnp.int32)


def _convert(utT, itT):
    eu = jnp.eye(EMBED, 128, dtype=jnp.float32)
    ei = jnp.eye(EMBED, 128, k=EMBED, dtype=jnp.float32)
    grid = (math.ceil(NROWS / CVT_BM),)

    def full(shape):
        return pl.BlockSpec(shape, lambda i: (0, 0))

    return pl.pallas_call(
        _cvt_body,
        grid=grid,
        in_specs=[
            pl.BlockSpec((EMBED, CVT_BM), lambda i: (0, i)),
            pl.BlockSpec((EMBED, CVT_BM), lambda i: (0, i)),
            full((EMBED, 128)),
            full((EMBED, 128)),
        ],
        out_specs=pl.BlockSpec((CVT_BM // 2, 128), lambda i: (i, 0)),
        out_shape=jax.ShapeDtypeStruct((NROWS // 2, 128), jnp.int32),
        compiler_params=pltpu.CompilerParams(
            fuse_transposed_lhs_in_matmul=True,
            vmem_limit_bytes=100 * 1024 * 1024,
        ),
    )(utT, itT, eu, ei)


def _sc_gather(uids2d, iids2d, table):
    mesh = plsc.VectorSubcoreMesh(core_axis_name="c", subcore_axis_name="s")

    @functools.partial(
        pl.kernel,
        mesh=mesh,
        out_type=(
            jax.ShapeDtypeStruct((BATCH, 128), jnp.int32),
            jax.ShapeDtypeStruct((BATCH, 128), jnp.int32),
        ),
        scratch_types=[
            pltpu.VMEM((NCHUNK, IDX_CHUNK), jnp.int32),
            pltpu.VMEM((NCHUNK, IDX_CHUNK), jnp.int32),
            pltpu.VMEM((BPW, 128), jnp.int32),
            pltpu.SemaphoreType.DMA,
        ],
    )
    def gather_kernel(uidx_hbm, iidx_hbm, tbl_hbm, uout_hbm, iout_hbm,
                      uidx_v, iidx_v, rows_v, sem):
        wid = lax.axis_index("s") * 2 + lax.axis_index("c")
        base = wid * BPW
        idx_row0 = wid * NCHUNK
        pltpu.sync_copy(uidx_hbm.at[pl.ds(idx_row0, NCHUNK), :], uidx_v)
        pltpu.sync_copy(iidx_hbm.at[pl.ds(idx_row0, NCHUNK), :], iidx_v)
        for idx_v, out_hbm in ((uidx_v, uout_hbm), (iidx_v, iout_hbm)):
            copies = []
            for j in range(NCHUNK):
                dst = rows_v.at[pl.ds(j * IDX_CHUNK, IDX_CHUNK), :]
                copies.append(pltpu.async_copy(tbl_hbm.at[idx_v.at[j]], dst, sem))
            for c in copies:
                c.wait()
            pltpu.sync_copy(rows_v, out_hbm.at[pl.ds(base, BPW), :])

    return gather_kernel(uids2d, iids2d, table)


def _unpack(w, par):
    odd = par > 0
    bits = jnp.where(odd, w & jnp.int32(-65536), w << 16)
    return pltpu.bitcast(bits, jnp.float32)


def _mlp_body(u_ref, i_ref, pu_ref, pi_ref, w1a_ref, w1b_ref, b1_ref,
              w2_ref, b2_ref, w3_ref, b3_ref, out_ref):
    ue = _unpack(u_ref[...], pu_ref[...])[:, :EMBED]
    ie = _unpack(i_ref[...], pi_ref[...])[:, EMBED:]
    h1 = jnp.dot(ue, w1a_ref[...], preferred_element_type=jnp.float32)
    h1 += jnp.dot(ie, w1b_ref[...], preferred_element_type=jnp.float32)
    h1 = jnp.maximum(h1 + b1_ref[...], 0.0)
    h2 = jnp.dot(h1, w2_ref[...], preferred_element_type=jnp.float32)
    h2 = jnp.maximum(h2 + b2_ref[...], 0.0)
    logit = jnp.dot(h2, w3_ref[...], preferred_element_type=jnp.float32)
    logit = logit + b3_ref[...]
    out_ref[...] = 1.0 / (1.0 + jnp.exp(-logit))


def _mlp(urows, irows, upar, ipar, w1a, w1b, b1, w2, b2, w3, b3,
         interpret=False):
    BM = 2048
    grid = (BATCH // BM,)

    def full(shape):
        return pl.BlockSpec(shape, lambda i: (0, 0))

    return pl.pallas_call(
        _mlp_body,
        grid=grid,
        in_specs=[
            pl.BlockSpec((BM, 128), lambda i: (i, 0)),
            pl.BlockSpec((BM, 128), lambda i: (i, 0)),
            pl.BlockSpec((BM, 1), lambda i: (i, 0)),
            pl.BlockSpec((BM, 1), lambda i: (i, 0)),
            full((EMBED, 128)),
            full((EMBED, 128)),
            full((1, 128)),
            full((128, EMBED)),
            full((1, EMBED)),
            full((EMBED, 1)),
            full((1, 1)),
        ],
        out_specs=pl.BlockSpec((BM, 1), lambda i: (i, 0)),
        out_shape=jax.ShapeDtypeStruct((BATCH, 1), jnp.float32),
        interpret=interpret,
    )(urows, irows, upar, ipar, w1a, w1b, b1, w2, b2, w3, b3)


def kernel(user_ids, item_ids, user_table, item_table, W1, b1, W2, b2, W3, b3):
    uids = user_ids.astype(jnp.int32)
    iids = item_ids.astype(jnp.int32)
    uidx2d = (uids >> 1).reshape(BATCH // IDX_CHUNK, IDX_CHUNK)
    iidx2d = (iids >> 1).reshape(BATCH // IDX_CHUNK, IDX_CHUNK)
    upar = (uids & 1).reshape(BATCH, 1)
    ipar = (iids & 1).reshape(BATCH, 1)
    table = _convert(user_table.T, item_table.T)
    urows, irows = _sc_gather(uidx2d, iidx2d, table)
    w1a = W1[:, :EMBED].T
    w1b = W1[:, EMBED:].T
    return _mlp(urows, irows, upar, ipar, w1a, w1b, b1.reshape(1, 128),
                W2.T, b2.reshape(1, EMBED), W3.T, b3.reshape(1, 1))


# speedup vs baseline: 3.3255x; 1.0003x over previous
import functools
import math

import jax
import jax.numpy as jnp
from jax import lax
from jax.experimental import pallas as pl
from jax.experimental.pallas import tpu as pltpu
from jax.experimental.pallas import tpu_sc as plsc

BATCH = 16384
EMBED = 64
NROWS = 1000000
NWORKERS = 32
BPW = BATCH // NWORKERS
IDX_CHUNK = 128
NCHUNK = BPW // IDX_CHUNK
CVT_BM = 32768


def _cvt_body(u_ref, i_ref, eu_ref, ei_ref, o_ref):
    dn = (((0,), (0,)), ((), ()))
    c = (
        lax.dot_general(u_ref[...], eu_ref[...], dn,
                        preferred_element_type=jnp.float32)
        + lax.dot_general(i_ref[...], ei_ref[...], dn,
                          preferred_element_type=jnp.float32)
    )
    o_ref[...] = pltpu.bitcast(c.astype(jnp.bfloat16), jnp.int32)


def _convert(utT, itT):
    eu = jnp.eye(EMBED, 128, dtype=jnp.float32)
    ei = jnp.eye(EMBED, 128, k=EMBED, dtype=jnp.float32)
    grid = (math.ceil(NROWS / CVT_BM),)

    def full(shape):
        return pl.BlockSpec(shape, lambda i: (0, 0))

    return pl.pallas_call(
        _cvt_body,
        grid=grid,
        in_specs=[
            pl.BlockSpec((EMBED, CVT_BM), lambda i: (0, i)),
            pl.BlockSpec((EMBED, CVT_BM), lambda i: (0, i)),
            full((EMBED, 128)),
            full((EMBED, 128)),
        ],
        out_specs=pl.BlockSpec((CVT_BM // 2, 128), lambda i: (i, 0)),
        out_shape=jax.ShapeDtypeStruct((NROWS // 2, 128), jnp.int32),
        compiler_params=pltpu.CompilerParams(
            fuse_transposed_lhs_in_matmul=True,
            vmem_limit_bytes=100 * 1024 * 1024,
        ),
    )(utT, itT, eu, ei)


def _sc_gather(uids2d, iids2d, table):
    mesh = plsc.VectorSubcoreMesh(core_axis_name="c", subcore_axis_name="s")

    @functools.partial(
        pl.kernel,
        mesh=mesh,
        out_type=(
            jax.ShapeDtypeStruct((BATCH, 128), jnp.int32),
            jax.ShapeDtypeStruct((BATCH, 128), jnp.int32),
        ),
        scratch_types=[
            pltpu.VMEM((NCHUNK, IDX_CHUNK), jnp.int32),
            pltpu.VMEM((NCHUNK, IDX_CHUNK), jnp.int32),
            pltpu.VMEM((4 * IDX_CHUNK, 128), jnp.int32),
            [pltpu.SemaphoreType.DMA] * 4,
            [pltpu.SemaphoreType.DMA] * 4,
        ],
    )
    def gather_kernel(uidx_hbm, iidx_hbm, tbl_hbm, uout_hbm, iout_hbm,
                      uidx_v, iidx_v, rows_v, gsems, osems):
        wid = lax.axis_index("s") * 2 + lax.axis_index("c")
        base = wid * BPW
        idx_row0 = wid * NCHUNK
        pltpu.sync_copy(uidx_hbm.at[pl.ds(idx_row0, NCHUNK), :], uidx_v)
        pltpu.sync_copy(iidx_hbm.at[pl.ds(idx_row0, NCHUNK), :], iidx_v)
        tasks = ([(uidx_v, uout_hbm, j) for j in range(NCHUNK)]
                 + [(iidx_v, iout_hbm, j) for j in range(NCHUNK)])

        def fire_gather(t):
            idx_v, _, j = tasks[t]
            s = t % 4
            dst = rows_v.at[pl.ds(s * IDX_CHUNK, IDX_CHUNK), :]
            return pltpu.async_copy(tbl_hbm.at[idx_v.at[j]], dst, gsems[s])

        def fire_copyout(t):
            _, out_hbm, j = tasks[t]
            s = t % 4
            src = rows_v.at[pl.ds(s * IDX_CHUNK, IDX_CHUNK), :]
            dst = out_hbm.at[pl.ds(base + j * IDX_CHUNK, IDX_CHUNK), :]
            return pltpu.async_copy(src, dst, osems[s])

        gc, oc = {}, {}
        for t in range(8):
            if t >= 4:
                oc[t - 4].wait()
            gc[t] = fire_gather(t)
            if t >= 1:
                gc[t - 1].wait()
                oc[t - 1] = fire_copyout(t - 1)
        gc[7].wait()
        oc[7] = fire_copyout(7)
        for t in range(4, 8):
            oc[t].wait()

    return gather_kernel(uids2d, iids2d, table)


def _unpack(w, par):
    odd = par > 0
    bits = jnp.where(odd, w & jnp.int32(-65536), w << 16)
    return pltpu.bitcast(bits, jnp.float32)


def _mlp_body(u_ref, i_ref, pu_ref, pi_ref, w1a_ref, w1b_ref, b1_ref,
              w2_ref, b2_ref, w3_ref, b3_ref, out_ref):
    ue = _unpack(u_ref[...], pu_ref[...])[:, :EMBED]
    ie = _unpack(i_ref[...], pi_ref[...])[:, EMBED:]
    h1 = jnp.dot(ue, w1a_ref[...], preferred_element_type=jnp.float32)
    h1 += jnp.dot(ie, w1b_ref[...], preferred_element_type=jnp.float32)
    h1 = jnp.maximum(h1 + b1_ref[...], 0.0)
    h2 = jnp.dot(h1, w2_ref[...], preferred_element_type=jnp.float32)
    h2 = jnp.maximum(h2 + b2_ref[...], 0.0)
    logit = jnp.dot(h2, w3_ref[...], preferred_element_type=jnp.float32)
    logit = logit + b3_ref[...]
    out_ref[...] = 1.0 / (1.0 + jnp.exp(-logit))


def _mlp(urows, irows, upar, ipar, w1a, w1b, b1, w2, b2, w3, b3,
         interpret=False):
    BM = 2048
    grid = (BATCH // BM,)

    def full(shape):
        return pl.BlockSpec(shape, lambda i: (0, 0))

    return pl.pallas_call(
        _mlp_body,
        grid=grid,
        in_specs=[
            pl.BlockSpec((BM, 128), lambda i: (i, 0)),
            pl.BlockSpec((BM, 128), lambda i: (i, 0)),
            pl.BlockSpec((BM, 1), lambda i: (i, 0)),
            pl.BlockSpec((BM, 1), lambda i: (i, 0)),
            full((EMBED, 128)),
            full((EMBED, 128)),
            full((1, 128)),
            full((128, EMBED)),
            full((1, EMBED)),
            full((EMBED, 1)),
            full((1, 1)),
        ],
        out_specs=pl.BlockSpec((BM, 1), lambda i: (i, 0)),
        out_shape=jax.ShapeDtypeStruct((BATCH, 1), jnp.float32),
        interpret=interpret,
    )(urows, irows, upar, ipar, w1a, w1b, b1, w2, b2, w3, b3)


def kernel(user_ids, item_ids, user_table, item_table, W1, b1, W2, b2, W3, b3):
    uids = user_ids.astype(jnp.int32)
    iids = item_ids.astype(jnp.int32)
    uidx2d = (uids >> 1).reshape(BATCH // IDX_CHUNK, IDX_CHUNK)
    iidx2d = (iids >> 1).reshape(BATCH // IDX_CHUNK, IDX_CHUNK)
    upar = (uids & 1).reshape(BATCH, 1)
    ipar = (iids & 1).reshape(BATCH, 1)
    table = _convert(user_table.T, item_table.T)
    urows, irows = _sc_gather(uidx2d, iidx2d, table)
    w1a = W1[:, :EMBED].T
    w1b = W1[:, EMBED:].T
    return _mlp(urows, irows, upar, ipar, w1a, w1b, b1.reshape(1, 128),
                W2.T, b2.reshape(1, EMBED), W3.T, b3.reshape(1, 1))


# speedup vs baseline: 3.3392x; 1.0041x over previous
import functools
import math

import jax
import jax.numpy as jnp
from jax import lax
from jax.experimental import pallas as pl
from jax.experimental.pallas import tpu as pltpu
from jax.experimental.pallas import tpu_sc as plsc

BATCH = 16384
EMBED = 64
NROWS = 1000000
NWORKERS = 32
BPW = BATCH // NWORKERS
IDX_CHUNK = 128
NCHUNK = BPW // IDX_CHUNK
CVT_BM = 32768


def _cvt_body(u_ref, i_ref, eu_ref, ei_ref, o_ref):
    dn = (((0,), (0,)), ((), ()))
    c = (
        lax.dot_general(u_ref[...], eu_ref[...], dn,
                        preferred_element_type=jnp.float32)
        + lax.dot_general(i_ref[...], ei_ref[...], dn,
                          preferred_element_type=jnp.float32)
    )
    o_ref[...] = pltpu.bitcast(c.astype(jnp.bfloat16), jnp.int32)


def _convert(utT, itT):
    eu = jnp.eye(EMBED, 128, dtype=jnp.float32)
    ei = jnp.eye(EMBED, 128, k=EMBED, dtype=jnp.float32)
    grid = (math.ceil(NROWS / CVT_BM),)

    def full(shape):
        return pl.BlockSpec(shape, lambda i: (0, 0))

    return pl.pallas_call(
        _cvt_body,
        grid=grid,
        in_specs=[
            pl.BlockSpec((EMBED, CVT_BM), lambda i: (0, i)),
            pl.BlockSpec((EMBED, CVT_BM), lambda i: (0, i)),
            full((EMBED, 128)),
            full((EMBED, 128)),
        ],
        out_specs=pl.BlockSpec((CVT_BM // 2, 128), lambda i: (i, 0)),
        out_shape=jax.ShapeDtypeStruct((NROWS // 2, 128), jnp.int32),
        compiler_params=pltpu.CompilerParams(
            fuse_transposed_lhs_in_matmul=True,
            vmem_limit_bytes=100 * 1024 * 1024,
        ),
    )(utT, itT, eu, ei)


def _sc_gather(uids2d, iids2d, table):
    mesh = plsc.VectorSubcoreMesh(core_axis_name="c", subcore_axis_name="s")

    @functools.partial(
        pl.kernel,
        mesh=mesh,
        out_type=(
            jax.ShapeDtypeStruct((BATCH, 128), jnp.int32),
            jax.ShapeDtypeStruct((BATCH, 128), jnp.int32),
        ),
        scratch_types=[
            pltpu.VMEM((NCHUNK, IDX_CHUNK), jnp.int32),
            pltpu.VMEM((NCHUNK, IDX_CHUNK), jnp.int32),
            pltpu.VMEM((BPW, 128), jnp.int32),
            pltpu.SemaphoreType.DMA,
        ],
    )
    def gather_kernel(uidx_hbm, iidx_hbm, tbl_hbm, uout_hbm, iout_hbm,
                      uidx_v, iidx_v, rows_v, sem):
        wid = lax.axis_index("s") * 2 + lax.axis_index("c")
        base = wid * BPW
        idx_row0 = wid * NCHUNK
        pltpu.sync_copy(uidx_hbm.at[pl.ds(idx_row0, NCHUNK), :], uidx_v)
        pltpu.sync_copy(iidx_hbm.at[pl.ds(idx_row0, NCHUNK), :], iidx_v)
        for idx_v, out_hbm in ((uidx_v, uout_hbm), (iidx_v, iout_hbm)):
            copies = []
            for j in range(NCHUNK):
                dst = rows_v.at[pl.ds(j * IDX_CHUNK, IDX_CHUNK), :]
                copies.append(pltpu.async_copy(tbl_hbm.at[idx_v.at[j]], dst, sem))
            for c in copies:
                c.wait()
            pltpu.sync_copy(rows_v, out_hbm.at[pl.ds(base, BPW), :])

    return gather_kernel(uids2d, iids2d, table)


def _unpack(w, par):
    odd = par > 0
    bits = jnp.where(odd, w & jnp.int32(-65536), w << 16)
    return pltpu.bitcast(bits, jnp.float32)


def _mlp_body(u_ref, i_ref, pu_ref, pi_ref, w1a_ref, w1b_ref, b1_ref,
              w2_ref, b2_ref, w3_ref, b3_ref, out_ref):
    ue = _unpack(u_ref[...], pu_ref[...])[:, :EMBED]
    ie = _unpack(i_ref[...], pi_ref[...])[:, EMBED:]
    h1 = jnp.dot(ue, w1a_ref[...], preferred_element_type=jnp.float32)
    h1 += jnp.dot(ie, w1b_ref[...], preferred_element_type=jnp.float32)
    h1 = jnp.maximum(h1 + b1_ref[...], 0.0)
    h2 = jnp.dot(h1, w2_ref[...], preferred_element_type=jnp.float32)
    h2 = jnp.maximum(h2 + b2_ref[...], 0.0)
    logit = jnp.dot(h2, w3_ref[...], preferred_element_type=jnp.float32)
    logit = logit + b3_ref[...]
    out_ref[...] = 1.0 / (1.0 + jnp.exp(-logit))


def _mlp(urows, irows, upar, ipar, w1a, w1b, b1, w2, b2, w3, b3,
         interpret=False):
    BM = 2048
    grid = (BATCH // BM,)

    def full(shape):
        return pl.BlockSpec(shape, lambda i: (0, 0))

    return pl.pallas_call(
        _mlp_body,
        grid=grid,
        in_specs=[
            pl.BlockSpec((BM, 128), lambda i: (i, 0)),
            pl.BlockSpec((BM, 128), lambda i: (i, 0)),
            pl.BlockSpec((BM, 1), lambda i: (i, 0)),
            pl.BlockSpec((BM, 1), lambda i: (i, 0)),
            full((EMBED, 128)),
            full((EMBED, 128)),
            full((1, 128)),
            full((128, EMBED)),
            full((1, EMBED)),
            full((EMBED, 1)),
            full((1, 1)),
        ],
        out_specs=pl.BlockSpec((BM, 1), lambda i: (i, 0)),
        out_shape=jax.ShapeDtypeStruct((BATCH, 1), jnp.float32),
        interpret=interpret,
    )(urows, irows, upar, ipar, w1a, w1b, b1, w2, b2, w3, b3)


def kernel(user_ids, item_ids, user_table, item_table, W1, b1, W2, b2, W3, b3):
    uids = user_ids.astype(jnp.int32)
    iids = item_ids.astype(jnp.int32)
    uidx2d = (uids >> 1).reshape(BATCH // IDX_CHUNK, IDX_CHUNK)
    iidx2d = (iids >> 1).reshape(BATCH // IDX_CHUNK, IDX_CHUNK)
    upar = (uids & 1).reshape(BATCH, 1)
    ipar = (iids & 1).reshape(BATCH, 1)
    table = _convert(user_table.T, item_table.T)
    urows, irows = _sc_gather(uidx2d, iidx2d, table)
    w1a = W1[:, :EMBED].T
    w1b = W1[:, EMBED:].T
    return _mlp(urows, irows, upar, ipar, w1a, w1b, b1.reshape(1, 128),
                W2.T, b2.reshape(1, EMBED), W3.T, b3.reshape(1, 1))
